# Initial kernel scaffold; baseline (speedup 1.0000x reference)
import jax
import jax.numpy as jnp
from jax.experimental import pallas as pl


def kernel(edge_index, node_emb, Wl1, Wr1, b1, Wl2, Wr2, b2):
    raise NotImplementedError("write your pallas kernel here")


# speedup vs baseline: 3.6268x; 3.6268x over previous
import functools

import jax
import jax.numpy as jnp
from jax import lax
from jax.experimental import pallas as pl
from jax.experimental.pallas import tpu as pltpu
from jax.experimental.pallas import tpu_sc as plsc

N_NODES = 10000
HIDDEN = 128
N_EDGES = 320000

NC = 2
NS = 16
NW = NC * NS
CHUNK = 128
K = 80
SB = 5
KSB = K // SB
E_PAD = NW * K * CHUNK
ACC = 10240
ROWS_PER_TILE = ACC // NS
DUMMY_DST = N_NODES

_mesh = plsc.VectorSubcoreMesh(core_axis_name="c", subcore_axis_name="s")


def _make_sc_agg():
    out_type = (jax.ShapeDtypeStruct((NC, ACC, HIDDEN), jnp.float32),
                jax.ShapeDtypeStruct((NC, ACC), jnp.float32))

    scratch = [
        pltpu.VMEM((KSB, CHUNK), jnp.int32),
        pltpu.VMEM((KSB, CHUNK), jnp.int32),
        pltpu.VMEM((2, CHUNK, HIDDEN), jnp.float32),
        pltpu.VMEM((CHUNK,), jnp.float32),
        pltpu.VMEM_SHARED((ACC, HIDDEN), jnp.float32),
        pltpu.VMEM_SHARED((ACC,), jnp.float32),
        pltpu.SemaphoreType.DMA,
        pltpu.SemaphoreType.DMA,
    ]

    @functools.partial(
        pl.kernel,
        out_type=out_type,
        mesh=_mesh,
        scratch_types=scratch,
    )
    def sc_agg(y_hbm, src_hbm, dst_hbm, out_hbm, deg_hbm,
               src_v, dst_v, rows_v, ones_v, acc_s, deg_s,
               sem0, sem1):

        c = lax.axis_index("c")
        s = lax.axis_index("s")
        wid = s * NC + c

        zeros16 = jnp.zeros((16,), jnp.float32)
        ones16 = jnp.ones((16,), jnp.float32)

        def zrow(r, carry):
            for q in range(HIDDEN // 16):
                rows_v[0, r, pl.ds(q * 16, 16)] = zeros16
            return carry

        lax.fori_loop(0, CHUNK, zrow, 0)
        for q in range(CHUNK // 16):
            ones_v[pl.ds(q * 16, 16)] = ones16

        base = s * ROWS_PER_TILE
        for k5 in range(ROWS_PER_TILE // CHUNK):
            pltpu.sync_copy(rows_v.at[0],
                            acc_s.at[pl.ds(base + k5 * CHUNK, CHUNK)])
            pltpu.sync_copy(rows_v.at[0, 0],
                            deg_s.at[pl.ds(base + k5 * CHUNK, CHUNK)])

        plsc.subcore_barrier()

        def sb_body(sb, carry):
            pltpu.sync_copy(src_hbm.at[wid, pl.ds(sb * KSB, KSB)], src_v)
            pltpu.sync_copy(dst_hbm.at[wid, pl.ds(sb * KSB, KSB)], dst_v)

            pltpu.async_copy(y_hbm.at[src_v.at[0]], rows_v.at[0], sem0)
            pltpu.async_copy(y_hbm.at[src_v.at[1]], rows_v.at[1], sem1)

            def pair(p, c2):
                j0 = p * 2
                for b, sem in ((0, sem0), (1, sem1)):
                    j = j0 + b
                    pltpu.make_async_copy(
                        y_hbm.at[src_v.at[j]], rows_v.at[b], sem).wait()
                    pltpu.sync_copy(rows_v.at[b], acc_s.at[dst_v.at[j]],
                                    add=True)
                    pltpu.sync_copy(ones_v, deg_s.at[dst_v.at[j]], add=True)

                    @pl.when(j + 2 < KSB)
                    def _():
                        pltpu.async_copy(
                            y_hbm.at[src_v.at[j + 2]], rows_v.at[b], sem)

                return c2

            lax.fori_loop(0, KSB // 2, pair, 0)
            return carry

        lax.fori_loop(0, SB, sb_body, 0)

        plsc.subcore_barrier()

        pltpu.sync_copy(acc_s.at[pl.ds(base, ROWS_PER_TILE)],
                        out_hbm.at[c, pl.ds(base, ROWS_PER_TILE)])
        pltpu.sync_copy(deg_s.at[pl.ds(base, ROWS_PER_TILE)],
                        deg_hbm.at[c, pl.ds(base, ROWS_PER_TILE)])

    return sc_agg


_sc_agg = _make_sc_agg()

BLK = 2000
_DN = (((1,), (1,)), ((), ()))


def _mm2_body(x_ref, wl_ref, wr_ref, y_ref, z_ref):
    x = x_ref[...]
    y_ref[...] = lax.dot_general(x, wl_ref[...], _DN,
                                 preferred_element_type=jnp.float32)
    z_ref[...] = lax.dot_general(x, wr_ref[...], _DN,
                                 preferred_element_type=jnp.float32)


def _mid_body(aggp_ref, degp_ref, z1_ref, b1_ref, wl2_ref, wr2_ref,
              y2_ref, z2_ref):
    p = aggp_ref[0] + aggp_ref[1]
    deg = degp_ref[0, :, 0] + degp_ref[1, :, 0]
    rdeg = 1.0 / jnp.maximum(deg, 1.0)
    x1 = jnp.maximum(p * rdeg[:, None] + b1_ref[...][None, :] + z1_ref[...],
                     0.0)
    y2_ref[...] = lax.dot_general(x1, wl2_ref[...], _DN,
                                  preferred_element_type=jnp.float32)
    z2_ref[...] = lax.dot_general(x1, wr2_ref[...], _DN,
                                  preferred_element_type=jnp.float32)


def _fin_body(aggp_ref, degp_ref, z2_ref, b2_ref, out_ref):
    p = aggp_ref[0] + aggp_ref[1]
    deg = degp_ref[0, :, 0] + degp_ref[1, :, 0]
    rdeg = 1.0 / jnp.maximum(deg, 1.0)
    out_ref[...] = p * rdeg[:, None] + b2_ref[...][None, :] + z2_ref[...]


_w_spec = pl.BlockSpec((HIDDEN, HIDDEN), lambda i: (0, 0))
_b_spec = pl.BlockSpec((HIDDEN,), lambda i: (0,))
_row_spec = pl.BlockSpec((BLK, HIDDEN), lambda i: (i, 0))
_aggp_spec = pl.BlockSpec((NC, BLK, HIDDEN), lambda i: (0, i, 0))
_degp_spec = pl.BlockSpec((NC, BLK, 1), lambda i: (0, i, 0))
_row_shape = jax.ShapeDtypeStruct((N_NODES, HIDDEN), jnp.float32)

_mm2 = pl.pallas_call(
    _mm2_body,
    grid=(N_NODES // BLK,),
    in_specs=[_row_spec, _w_spec, _w_spec],
    out_specs=[_row_spec, _row_spec],
    out_shape=[_row_shape, _row_shape],
)

_mid = pl.pallas_call(
    _mid_body,
    grid=(N_NODES // BLK,),
    in_specs=[_aggp_spec, _degp_spec, _row_spec, _b_spec, _w_spec, _w_spec],
    out_specs=[_row_spec, _row_spec],
    out_shape=[_row_shape, _row_shape],
)

_fin = pl.pallas_call(
    _fin_body,
    grid=(N_NODES // BLK,),
    in_specs=[_aggp_spec, _degp_spec, _row_spec, _b_spec],
    out_specs=_row_spec,
    out_shape=_row_shape,
)


def kernel(edge_index, node_emb, Wl1, Wr1, b1, Wl2, Wr2, b2):
    pad = E_PAD - N_EDGES
    src = jnp.concatenate(
        [edge_index[0], jnp.zeros((pad,), jnp.int32)]).reshape(NW, K, CHUNK)
    dst = jnp.concatenate(
        [edge_index[1],
         jnp.full((pad,), DUMMY_DST, jnp.int32)]).reshape(NW, K, CHUNK)

    y1, z1 = _mm2(node_emb, Wl1, Wr1)
    aggp1, degp = _sc_agg(y1, src, dst)
    degp = degp.reshape(NC, ACC, 1)
    y2, z2 = _mid(aggp1, degp, z1, b1, Wl2, Wr2)
    aggp2, _unused_deg = _sc_agg(y2, src, dst)
    return _fin(aggp2, degp, z2, b2)


# speedup vs baseline: 3.6837x; 1.0157x over previous
import functools

import jax
import jax.numpy as jnp
from jax import lax
from jax.experimental import pallas as pl
from jax.experimental.pallas import tpu as pltpu
from jax.experimental.pallas import tpu_sc as plsc

N_NODES = 10000
HIDDEN = 128
N_EDGES = 320000

NC = 2
NS = 16
NW = NC * NS
CHUNK = 128
K = 80
SB = 5
KSB = K // SB
E_PAD = NW * K * CHUNK
ACC = 10240
ROWS_PER_TILE = ACC // NS
DUMMY_DST = N_NODES

_mesh = plsc.VectorSubcoreMesh(core_axis_name="c", subcore_axis_name="s")


def _make_sc_agg():
    out_type = (jax.ShapeDtypeStruct((NC, ACC, HIDDEN), jnp.float32),
                jax.ShapeDtypeStruct((NC, ACC), jnp.float32))

    scratch = [
        pltpu.VMEM((KSB, CHUNK), jnp.int32),
        pltpu.VMEM((KSB, CHUNK), jnp.int32),
        pltpu.VMEM((2, CHUNK, HIDDEN), jnp.float32),
        pltpu.VMEM((CHUNK,), jnp.float32),
        pltpu.VMEM_SHARED((ACC, HIDDEN), jnp.float32),
        pltpu.VMEM_SHARED((ACC,), jnp.float32),
        pltpu.SemaphoreType.DMA,
        pltpu.SemaphoreType.DMA,
        pltpu.SemaphoreType.DMA,
        pltpu.SemaphoreType.DMA,
        pltpu.SemaphoreType.DMA,
    ]

    @functools.partial(
        pl.kernel,
        out_type=out_type,
        mesh=_mesh,
        scratch_types=scratch,
    )
    def sc_agg(y_hbm, src_hbm, dst_hbm, out_hbm, deg_hbm,
               src_v, dst_v, rows_v, ones_v, acc_s, deg_s,
               semg0, semg1, sems0, sems1, semd):

        c = lax.axis_index("c")
        s = lax.axis_index("s")
        wid = s * NC + c

        zeros16 = jnp.zeros((16,), jnp.float32)
        ones16 = jnp.ones((16,), jnp.float32)

        def zrow(r, carry):
            for q in range(HIDDEN // 16):
                rows_v[0, r, pl.ds(q * 16, 16)] = zeros16
            return carry

        lax.fori_loop(0, CHUNK, zrow, 0)
        for q in range(CHUNK // 16):
            ones_v[pl.ds(q * 16, 16)] = ones16

        base = s * ROWS_PER_TILE
        for k5 in range(ROWS_PER_TILE // CHUNK):
            pltpu.sync_copy(rows_v.at[0],
                            acc_s.at[pl.ds(base + k5 * CHUNK, CHUNK)])
            pltpu.sync_copy(rows_v.at[0, 0],
                            deg_s.at[pl.ds(base + k5 * CHUNK, CHUNK)])

        plsc.subcore_barrier()

        def _wait_gather(j, b, sem):
            pltpu.make_async_copy(
                y_hbm.at[src_v.at[j]], rows_v.at[b], sem).wait()

        def _wait_scatter(j, b, sem):
            pltpu.make_async_copy(
                rows_v.at[b], acc_s.at[dst_v.at[j]], sem).wait()

        def _wait_deg(j):
            pltpu.make_async_copy(
                ones_v, deg_s.at[dst_v.at[j]], semd).wait()

        def sb_body(sb, carry):
            pltpu.sync_copy(src_hbm.at[wid, pl.ds(sb * KSB, KSB)], src_v)
            pltpu.sync_copy(dst_hbm.at[wid, pl.ds(sb * KSB, KSB)], dst_v)

            pltpu.async_copy(y_hbm.at[src_v.at[0]], rows_v.at[0], semg0)

            def pair(p, c2):
                j0 = p * 2
                _wait_gather(j0, 0, semg0)
                pltpu.async_copy(rows_v.at[0], acc_s.at[dst_v.at[j0]], sems0,
                                 add=True)
                pltpu.async_copy(ones_v, deg_s.at[dst_v.at[j0]], semd,
                                 add=True)

                @pl.when(p > 0)
                def _():
                    _wait_scatter(j0 - 1, 1, sems1)
                    _wait_deg(j0 - 1)

                pltpu.async_copy(y_hbm.at[src_v.at[j0 + 1]], rows_v.at[1],
                                 semg1)

                _wait_gather(j0 + 1, 1, semg1)
                pltpu.async_copy(rows_v.at[1], acc_s.at[dst_v.at[j0 + 1]],
                                 sems1, add=True)
                pltpu.async_copy(ones_v, deg_s.at[dst_v.at[j0 + 1]], semd,
                                 add=True)
                _wait_scatter(j0, 0, sems0)
                _wait_deg(j0)

                @pl.when(p < KSB // 2 - 1)
                def _():
                    pltpu.async_copy(y_hbm.at[src_v.at[j0 + 2]], rows_v.at[0],
                                     semg0)

                return c2

            lax.fori_loop(0, KSB // 2, pair, 0)
            _wait_scatter(KSB - 1, 1, sems1)
            _wait_deg(KSB - 1)
            return carry

        lax.fori_loop(0, SB, sb_body, 0)

        plsc.subcore_barrier()

        pltpu.sync_copy(acc_s.at[pl.ds(base, ROWS_PER_TILE)],
                        out_hbm.at[c, pl.ds(base, ROWS_PER_TILE)])
        pltpu.sync_copy(deg_s.at[pl.ds(base, ROWS_PER_TILE)],
                        deg_hbm.at[c, pl.ds(base, ROWS_PER_TILE)])

    return sc_agg


_sc_agg = _make_sc_agg()

BLK = 2000
_DN = (((1,), (1,)), ((), ()))


def _mm2_body(x_ref, wl_ref, wr_ref, y_ref, z_ref):
    x = x_ref[...]
    y_ref[...] = lax.dot_general(x, wl_ref[...], _DN,
                                 preferred_element_type=jnp.float32)
    z_ref[...] = lax.dot_general(x, wr_ref[...], _DN,
                                 preferred_element_type=jnp.float32)


def _mid_body(aggp_ref, degp_ref, z1_ref, b1_ref, wl2_ref, wr2_ref,
              y2_ref, z2_ref):
    p = aggp_ref[0] + aggp_ref[1]
    deg = degp_ref[0, :, 0] + degp_ref[1, :, 0]
    rdeg = 1.0 / jnp.maximum(deg, 1.0)
    x1 = jnp.maximum(p * rdeg[:, None] + b1_ref[...][None, :] + z1_ref[...],
                     0.0)
    y2_ref[...] = lax.dot_general(x1, wl2_ref[...], _DN,
                                  preferred_element_type=jnp.float32)
    z2_ref[...] = lax.dot_general(x1, wr2_ref[...], _DN,
                                  preferred_element_type=jnp.float32)


def _fin_body(aggp_ref, degp_ref, z2_ref, b2_ref, out_ref):
    p = aggp_ref[0] + aggp_ref[1]
    deg = degp_ref[0, :, 0] + degp_ref[1, :, 0]
    rdeg = 1.0 / jnp.maximum(deg, 1.0)
    out_ref[...] = p * rdeg[:, None] + b2_ref[...][None, :] + z2_ref[...]


_w_spec = pl.BlockSpec((HIDDEN, HIDDEN), lambda i: (0, 0))
_b_spec = pl.BlockSpec((HIDDEN,), lambda i: (0,))
_row_spec = pl.BlockSpec((BLK, HIDDEN), lambda i: (i, 0))
_aggp_spec = pl.BlockSpec((NC, BLK, HIDDEN), lambda i: (0, i, 0))
_degp_spec = pl.BlockSpec((NC, BLK, 1), lambda i: (0, i, 0))
_row_shape = jax.ShapeDtypeStruct((N_NODES, HIDDEN), jnp.float32)

_mm2 = pl.pallas_call(
    _mm2_body,
    grid=(N_NODES // BLK,),
    in_specs=[_row_spec, _w_spec, _w_spec],
    out_specs=[_row_spec, _row_spec],
    out_shape=[_row_shape, _row_shape],
)

_mid = pl.pallas_call(
    _mid_body,
    grid=(N_NODES // BLK,),
    in_specs=[_aggp_spec, _degp_spec, _row_spec, _b_spec, _w_spec, _w_spec],
    out_specs=[_row_spec, _row_spec],
    out_shape=[_row_shape, _row_shape],
)

_fin = pl.pallas_call(
    _fin_body,
    grid=(N_NODES // BLK,),
    in_specs=[_aggp_spec, _degp_spec, _row_spec, _b_spec],
    out_specs=_row_spec,
    out_shape=_row_shape,
)


def kernel(edge_index, node_emb, Wl1, Wr1, b1, Wl2, Wr2, b2):
    pad = E_PAD - N_EDGES
    src = jnp.concatenate(
        [edge_index[0], jnp.zeros((pad,), jnp.int32)]).reshape(NW, K, CHUNK)
    dst = jnp.concatenate(
        [edge_index[1],
         jnp.full((pad,), DUMMY_DST, jnp.int32)]).reshape(NW, K, CHUNK)

    y1, z1 = _mm2(node_emb, Wl1, Wr1)
    aggp1, degp = _sc_agg(y1, src, dst)
    degp = degp.reshape(NC, ACC, 1)
    y2, z2 = _mid(aggp1, degp, z1, b1, Wl2, Wr2)
    aggp2, _unused_deg = _sc_agg(y2, src, dst)
    return _fin(aggp2, degp, z2, b2)
